# baseline (speedup 1.0000x reference)
import functools

import numpy as np
import jax
import jax.numpy as jnp
from jax import lax
from jax.experimental import pallas as pl
from jax.experimental.pallas import tpu as pltpu
from jax.experimental.pallas import tpu_sc as plsc

_B = 128
_V = 100000
_C = 3584

_NB1 = 18
_M = _NB1 * _C
_NB2 = 10
_W = _NB2 * _C

_TINY = np.float32(np.finfo(np.float32).tiny)
_KS = (0, 1, 0x1BD11BDB)
_ROT = ((13, 15, 26, 6), (17, 29, 16, 24))


def _gumbel_from_bits(bits):
    fb = (bits >> jnp.uint32(9)) | jnp.uint32(0x3F800000)
    f = lax.bitcast_convert_type(fb, jnp.float32) - jnp.float32(1.0)
    u = jnp.maximum(f, _TINY)
    return -jnp.log(-jnp.log(u))


def _blockmax(v, col):
    m = jnp.max(v, axis=1, keepdims=True)
    cand = jnp.where(v == m, col, jnp.int32(0x7FFFFFFF))
    idx = jnp.min(cand, axis=1, keepdims=True)
    return m, idx


def _tc_main_body(x_ref, cnt_ref, bestv_ref, besti_ref):
    j = pl.program_id(0)

    @pl.when(j == 0)
    def _init():
        bestv_ref[...] = jnp.full((_B, 1), -jnp.inf, jnp.float32)
        besti_ref[...] = jnp.zeros((_B, 1), jnp.int32)

    x = x_ref[...]
    x1 = cnt_ref[...] + jnp.uint32(j * _C)
    x0 = jnp.zeros((_B, _C), jnp.uint32)
    for r in range(5):
        for rr in _ROT[r % 2]:
            x0 = x0 + x1
            x1 = (x1 << jnp.uint32(rr)) | (x1 >> jnp.uint32(32 - rr))
            x1 = x0 ^ x1
        x0 = x0 + jnp.uint32(_KS[(r + 1) % 3])
        x1 = x1 + jnp.uint32(_KS[(r + 2) % 3] + r + 1)
    bits = x0 ^ x1

    v = x + _gumbel_from_bits(bits)
    col = lax.broadcasted_iota(jnp.int32, (_B, _C), 1)
    m, idx = _blockmax(v, col)
    idx = idx + j * _C

    upd = m > bestv_ref[...]
    bestv_ref[...] = jnp.where(upd, m, bestv_ref[...])
    besti_ref[...] = jnp.where(upd, idx, besti_ref[...])


def _tc_main(logits, cnt0):
    return pl.pallas_call(
        _tc_main_body,
        grid=(_NB1,),
        in_specs=[
            pl.BlockSpec((_B, _C), lambda j: (0, j)),
            pl.BlockSpec((_B, _C), lambda j: (0, 0)),
        ],
        out_specs=[
            pl.BlockSpec((_B, 1), lambda j: (0, 0)),
            pl.BlockSpec((_B, 1), lambda j: (0, 0)),
        ],
        out_shape=[
            jax.ShapeDtypeStruct((_B, 1), jnp.float32),
            jax.ShapeDtypeStruct((_B, 1), jnp.int32),
        ],
    )(logits, cnt0)


_SC_UNROLL = 8


def _sc_threefry_vec(x1):
    x0 = jnp.zeros((16,), jnp.uint32)
    for r in range(5):
        for rr in _ROT[r % 2]:
            x0 = x0 + x1
            x1 = (x1 << jnp.uint32(rr)) | (x1 >> jnp.uint32(32 - rr))
            x1 = x0 ^ x1
        x0 = x0 + jnp.uint32(_KS[(r + 1) % 3])
        x1 = x1 + jnp.uint32(_KS[(r + 2) % 3] + r + 1)
    return x0 ^ x1


def _sc_bits_kernel(out_hbm, buf, sem):
    wid = lax.axis_index("s") * 2 + lax.axis_index("c")
    lane = lax.bitcast_convert_type(lax.iota(jnp.int32, 16), jnp.uint32)
    for rr in range(4):
        r = wid * 4 + rr
        base = jnp.uint32(r) * jnp.uint32(_V) + jnp.uint32(_M + 1)

        def body(i, carry):
            for k in range(_SC_UNROLL):
                off = i * (16 * _SC_UNROLL) + k * 16
                buf[pl.ds(off, 16)] = _sc_threefry_vec(
                    base + jnp.uint32(off) + lane)
            return carry

        lax.fori_loop(0, _W // (16 * _SC_UNROLL), body, jnp.int32(0))
        cp = pltpu.make_async_copy(buf, out_hbm.at[r], sem)
        cp.start()
        cp.wait()


def _sc_bits():
    mesh = plsc.VectorSubcoreMesh(core_axis_name="c", subcore_axis_name="s")
    return pl.kernel(
        _sc_bits_kernel,
        out_type=jax.ShapeDtypeStruct((_B, _W), jnp.uint32),
        mesh=mesh,
        scratch_types=[
            pltpu.VMEM((_W,), jnp.uint32),
            pltpu.SemaphoreType.DMA,
        ],
    )()


def _tc_tail_body(x_ref, bits_ref, bv_ref, bi_ref, out_ref,
                  bestv_ref, besti_ref):
    j = pl.program_id(0)

    @pl.when(j == 0)
    def _init():
        bestv_ref[...] = bv_ref[...]
        besti_ref[...] = bi_ref[...]

    v = x_ref[...] + _gumbel_from_bits(bits_ref[...])
    col = lax.broadcasted_iota(jnp.int32, (_B, _C), 1)
    v = jnp.where(col < _V - _M - j * _C, v, -jnp.inf)
    m, idx = _blockmax(v, col)
    idx = idx + (_M + j * _C)

    upd = m > bestv_ref[...]
    bestv_ref[...] = jnp.where(upd, m, bestv_ref[...])
    besti_ref[...] = jnp.where(upd, idx, besti_ref[...])

    @pl.when(j == _NB2 - 1)
    def _fin():
        out_ref[...] = besti_ref[...]


def _tc_tail(logits, bits, bv, bi):
    return pl.pallas_call(
        _tc_tail_body,
        grid=(_NB2,),
        in_specs=[
            pl.BlockSpec((_B, _C), lambda j: (0, _NB1 + j)),
            pl.BlockSpec((_B, _C), lambda j: (0, j)),
            pl.BlockSpec((_B, 1), lambda j: (0, 0)),
            pl.BlockSpec((_B, 1), lambda j: (0, 0)),
        ],
        out_specs=pl.BlockSpec((_B, 1), lambda j: (0, 0)),
        out_shape=jax.ShapeDtypeStruct((_B, 1), jnp.int32),
        scratch_shapes=[
            pltpu.VMEM((_B, 1), jnp.float32),
            pltpu.VMEM((_B, 1), jnp.int32),
        ],
    )(logits, bits, bv, bi)


def _base_counters():
    r = np.arange(_B, dtype=np.uint64)[:, None]
    c = np.arange(_C, dtype=np.uint64)[None, :]
    return jnp.asarray((r * _V + c + 1).astype(np.uint32))


def kernel(logits):
    bits = _sc_bits()
    bv, bi = _tc_main(logits, _base_counters())
    out = _tc_tail(logits, bits, bv, bi)
    return out.reshape(_B)


# speedup vs baseline: 1.0441x; 1.0441x over previous
import functools

import numpy as np
import jax
import jax.numpy as jnp
from jax import lax
from jax.experimental import pallas as pl
from jax.experimental.pallas import tpu as pltpu
from jax.experimental.pallas import tpu_sc as plsc

_B = 128
_V = 100000
_C = 3584

_NB1 = 19
_M = _NB1 * _C
_NB2 = 9
_W = _NB2 * _C

_TINY = np.float32(np.finfo(np.float32).tiny)
_KS = (0, 1, 0x1BD11BDB)
_ROT = ((13, 15, 26, 6), (17, 29, 16, 24))


def _gumbel_from_bits(bits):
    fb = (bits >> jnp.uint32(9)) | jnp.uint32(0x3F800000)
    f = lax.bitcast_convert_type(fb, jnp.float32) - jnp.float32(1.0)
    u = jnp.maximum(f, _TINY)
    return -jnp.log(-jnp.log(u))


def _blockmax(v, col):
    m = jnp.max(v, axis=1, keepdims=True)
    cand = jnp.where(v == m, col, jnp.int32(0x7FFFFFFF))
    idx = jnp.min(cand, axis=1, keepdims=True)
    return m, idx


def _tc_main_body(x_ref, cnt_ref, bestv_ref, besti_ref):
    j = pl.program_id(0)

    @pl.when(j == 0)
    def _init():
        bestv_ref[...] = jnp.full((_B, 1), -jnp.inf, jnp.float32)
        besti_ref[...] = jnp.zeros((_B, 1), jnp.int32)

    x = x_ref[...]
    x1 = cnt_ref[...] + jnp.uint32(j * _C)
    x0 = jnp.zeros((_B, _C), jnp.uint32)
    for r in range(5):
        for rr in _ROT[r % 2]:
            x0 = x0 + x1
            x1 = (x1 << jnp.uint32(rr)) | (x1 >> jnp.uint32(32 - rr))
            x1 = x0 ^ x1
        x0 = x0 + jnp.uint32(_KS[(r + 1) % 3])
        x1 = x1 + jnp.uint32(_KS[(r + 2) % 3] + r + 1)
    bits = x0 ^ x1

    v = x + _gumbel_from_bits(bits)
    col = lax.broadcasted_iota(jnp.int32, (_B, _C), 1)
    m, idx = _blockmax(v, col)
    idx = idx + j * _C

    upd = m > bestv_ref[...]
    bestv_ref[...] = jnp.where(upd, m, bestv_ref[...])
    besti_ref[...] = jnp.where(upd, idx, besti_ref[...])


def _tc_main(logits, cnt0):
    return pl.pallas_call(
        _tc_main_body,
        grid=(_NB1,),
        in_specs=[
            pl.BlockSpec((_B, _C), lambda j: (0, j)),
            pl.BlockSpec((_B, _C), lambda j: (0, 0)),
        ],
        out_specs=[
            pl.BlockSpec((_B, 1), lambda j: (0, 0)),
            pl.BlockSpec((_B, 1), lambda j: (0, 0)),
        ],
        out_shape=[
            jax.ShapeDtypeStruct((_B, 1), jnp.float32),
            jax.ShapeDtypeStruct((_B, 1), jnp.int32),
        ],
    )(logits, cnt0)


_SC_UNROLL = 8


def _sc_threefry_vec(x1):
    x0 = jnp.zeros((16,), jnp.uint32)
    for r in range(5):
        for rr in _ROT[r % 2]:
            x0 = x0 + x1
            x1 = (x1 << jnp.uint32(rr)) | (x1 >> jnp.uint32(32 - rr))
            x1 = x0 ^ x1
        x0 = x0 + jnp.uint32(_KS[(r + 1) % 3])
        x1 = x1 + jnp.uint32(_KS[(r + 2) % 3] + r + 1)
    return x0 ^ x1


def _sc_bits_kernel(out_hbm, buf, sem):
    wid = lax.axis_index("s") * 2 + lax.axis_index("c")
    lane = lax.bitcast_convert_type(lax.iota(jnp.int32, 16), jnp.uint32)
    for rr in range(4):
        r = wid * 4 + rr
        base = jnp.uint32(r) * jnp.uint32(_V) + jnp.uint32(_M + 1)

        def body(i, carry):
            for k in range(_SC_UNROLL):
                off = i * (16 * _SC_UNROLL) + k * 16
                buf[pl.ds(off, 16)] = _sc_threefry_vec(
                    base + jnp.uint32(off) + lane)
            return carry

        lax.fori_loop(0, _W // (16 * _SC_UNROLL), body, jnp.int32(0))
        cp = pltpu.make_async_copy(buf, out_hbm.at[r], sem)
        cp.start()
        cp.wait()


def _sc_bits():
    mesh = plsc.VectorSubcoreMesh(core_axis_name="c", subcore_axis_name="s")
    return pl.kernel(
        _sc_bits_kernel,
        out_type=jax.ShapeDtypeStruct((_B, _W), jnp.uint32),
        mesh=mesh,
        scratch_types=[
            pltpu.VMEM((_W,), jnp.uint32),
            pltpu.SemaphoreType.DMA,
        ],
    )()


def _tc_tail_body(x_ref, bits_ref, bv_ref, bi_ref, out_ref,
                  bestv_ref, besti_ref):
    j = pl.program_id(0)

    @pl.when(j == 0)
    def _init():
        bestv_ref[...] = bv_ref[...]
        besti_ref[...] = bi_ref[...]

    col = lax.broadcasted_iota(jnp.int32, (_B, _C), 1)
    v = x_ref[...] + _gumbel_from_bits(bits_ref[...])
    v = jnp.where(col < _V - _M - j * _C, v, -jnp.inf)
    m, idx = _blockmax(v, col)
    idx = idx + (_M + j * _C)

    upd = m > bestv_ref[...]
    bestv_ref[...] = jnp.where(upd, m, bestv_ref[...])
    besti_ref[...] = jnp.where(upd, idx, besti_ref[...])

    @pl.when(j == _NB2 - 1)
    def _fin():
        out_ref[...] = besti_ref[...]


def _tc_tail(logits, bits, bv, bi):
    return pl.pallas_call(
        _tc_tail_body,
        grid=(_NB2,),
        in_specs=[
            pl.BlockSpec((_B, _C), lambda j: (0, _NB1 + j)),
            pl.BlockSpec((_B, _C), lambda j: (0, j)),
            pl.BlockSpec((_B, 1), lambda j: (0, 0)),
            pl.BlockSpec((_B, 1), lambda j: (0, 0)),
        ],
        out_specs=pl.BlockSpec((_B, 1), lambda j: (0, 0)),
        out_shape=jax.ShapeDtypeStruct((_B, 1), jnp.int32),
        scratch_shapes=[
            pltpu.VMEM((_B, 1), jnp.float32),
            pltpu.VMEM((_B, 1), jnp.int32),
        ],
    )(logits, bits, bv, bi)


def _base_counters():
    r = np.arange(_B, dtype=np.uint64)[:, None]
    c = np.arange(_C, dtype=np.uint64)[None, :]
    return jnp.asarray((r * _V + c + 1).astype(np.uint32))


def kernel(logits):
    bits = _sc_bits()
    bv, bi = _tc_main(logits, _base_counters())
    out = _tc_tail(logits, bits, bv, bi)
    return out.reshape(_B)
